# Initial kernel scaffold; baseline (speedup 1.0000x reference)
import jax
import jax.numpy as jnp
from jax.experimental import pallas as pl


def kernel(x, rand_indices, rand_signs):
    raise NotImplementedError("write your pallas kernel here")


# speedup vs baseline: 13.7496x; 13.7496x over previous
import functools

import jax
import jax.numpy as jnp
from jax.experimental import pallas as pl
from jax.experimental.pallas import tpu as pltpu

BATCH = 1024
ORIG_DIM = 16384
PROJ_DIM = 1024
C = 4
KB = 2048
N_STEPS = ORIG_DIM // KB


def _body(x_ref, idx_ref, sgn_ref, o_ref, acc_ref):
    i = pl.program_id(0)

    @pl.when(i == 0)
    def _init():
        acc_ref[...] = jnp.zeros_like(acc_ref)

    idx = idx_ref[...]
    sgn = sgn_ref[...]
    iota = jax.lax.broadcasted_iota(jnp.int32, (KB, PROJ_DIM), 1)
    s = jnp.zeros((KB, PROJ_DIM), jnp.float32)
    for k in range(C):
        s = s + jnp.where(iota == idx[:, k:k + 1], sgn[:, k:k + 1], 0.0)
    xb = x_ref[...].astype(jnp.bfloat16)
    sb = s.astype(jnp.bfloat16)
    acc_ref[...] += jnp.dot(xb, sb, preferred_element_type=jnp.float32)

    @pl.when(i == N_STEPS - 1)
    def _done():
        o_ref[...] = acc_ref[...]


@jax.jit
def kernel(x, rand_indices, rand_signs):
    scaled_signs = rand_signs * (1.0 / jnp.sqrt(jnp.float32(C)))
    return pl.pallas_call(
        _body,
        grid=(N_STEPS,),
        in_specs=[
            pl.BlockSpec((BATCH, KB), lambda i: (0, i)),
            pl.BlockSpec((KB, C), lambda i: (i, 0)),
            pl.BlockSpec((KB, C), lambda i: (i, 0)),
        ],
        out_specs=pl.BlockSpec((BATCH, PROJ_DIM), lambda i: (0, 0)),
        out_shape=jax.ShapeDtypeStruct((BATCH, PROJ_DIM), jnp.float32),
        scratch_shapes=[pltpu.VMEM((BATCH, PROJ_DIM), jnp.float32)],
    )(x, rand_indices, scaled_signs)
